# baseline (speedup 1.0000x reference)
import jax
import jax.numpy as jnp
from jax import lax
from jax.experimental import pallas as pl
from jax.experimental.pallas import tpu as pltpu
from jax.experimental.pallas import tpu_sc as plsc

BATCH = 16384
N_PROTO = 256
N_WORKERS = 32
ROWS_PER_WORKER = BATCH // N_WORKERS
GROUP = 16
TILE = 64
N_TILES = ROWS_PER_WORKER // TILE
GROUPS_PER_TILE = TILE // GROUP
N_CHUNKS = N_PROTO // 16
BIG = 2**30


def _wtac_body(dist_hbm, labels_hbm, out_hbm, labels_v, tiles, out_v, sems):
    wid = lax.axis_index("c") * 16 + lax.axis_index("s")
    base = wid * ROWS_PER_WORKER

    pltpu.sync_copy(labels_hbm, labels_v)

    col_iota = lax.iota(jnp.int32, 16)

    def fire(t, b):
        pltpu.async_copy(
            dist_hbm.at[pl.ds(base + t * TILE, TILE), :], tiles[b], sems[b]
        )

    def drain(b):
        pltpu.make_async_copy(
            dist_hbm.at[pl.ds(0, TILE), :], tiles[b], sems[b]
        ).wait()

    for b in range(4):
        fire(b, b)

    def outer(o, carry):
        for b in range(4):
            t = 4 * o + b
            drain(b)
            tile = tiles[b]

            def group_body(gg, c2):
                r0 = gg * GROUP

                def half_body(h, labvec):
                    rbase = r0 + h * 8
                    for r in range(8):
                        row = rbase + r
                        best0 = tile[row, pl.ds(0, 16)]
                        bidx0 = jnp.zeros((16,), jnp.int32)
                        best1 = tile[row, pl.ds(128, 16)]
                        bidx1 = jnp.full((16,), 8, jnp.int32)
                        for j in range(1, 8):
                            v0 = tile[row, pl.ds(j * 16, 16)]
                            p0 = v0 < best0
                            best0 = jnp.where(p0, v0, best0)
                            bidx0 = jnp.where(
                                p0, jnp.full((16,), j, jnp.int32), bidx0
                            )
                            v1 = tile[row, pl.ds(128 + j * 16, 16)]
                            p1 = v1 < best1
                            best1 = jnp.where(p1, v1, best1)
                            bidx1 = jnp.where(
                                p1, jnp.full((16,), j + 8, jnp.int32), bidx1
                            )
                        pm = best1 < best0
                        best = jnp.where(pm, best1, best0)
                        bidx = jnp.where(pm, bidx1, bidx0)
                        m = jnp.min(best)
                        cand = jnp.where(
                            best == m,
                            bidx * 16 + col_iota,
                            jnp.full((16,), BIG, jnp.int32),
                        )
                        w = jnp.min(cand)
                        labvec = jnp.where(col_iota == (h * 8 + r), w, labvec)
                    return labvec

                labvec = lax.fori_loop(
                    0, 2, half_body, jnp.zeros((16,), jnp.int32)
                )
                out_v[pl.ds(t * TILE + r0, GROUP)] = plsc.load_gather(
                    labels_v, [labvec]
                )
                return c2

            lax.fori_loop(0, GROUPS_PER_TILE, group_body, 0)

            @pl.when(t + 4 < N_TILES)
            def _():
                fire(t + 4, b)
        return carry

    lax.fori_loop(0, N_TILES // 4, outer, 0)
    pltpu.sync_copy(out_v, out_hbm.at[pl.ds(base, ROWS_PER_WORKER)])


@jax.jit
def _wtac(distances, labels):
    mesh = plsc.VectorSubcoreMesh(core_axis_name="c", subcore_axis_name="s")
    run = pl.kernel(
        _wtac_body,
        out_type=jax.ShapeDtypeStruct((BATCH,), jnp.int32),
        mesh=mesh,
        scratch_types=[
            pltpu.VMEM((N_PROTO,), jnp.int32),
            [pltpu.VMEM((TILE, N_PROTO), jnp.float32)] * 4,
            pltpu.VMEM((ROWS_PER_WORKER,), jnp.int32),
            [pltpu.SemaphoreType.DMA] * 4,
        ],
        compiler_params=pltpu.CompilerParams(
            use_tc_tiling_on_sc=True, needs_layout_passes=False,
            disable_bounds_checks=True, disable_semaphore_checks=True,
            skip_device_barrier=True
        ),
        name="wtac_sc",
    )
    return run(distances, labels)


def kernel(distances, prototype_labels):
    labels = prototype_labels.astype(jnp.int32)
    return _wtac(distances, labels)


# speedup vs baseline: 1.1064x; 1.1064x over previous
import jax
import jax.numpy as jnp
from jax import lax
from jax.experimental import pallas as pl
from jax.experimental.pallas import tpu as pltpu
from jax.experimental.pallas import tpu_sc as plsc

BATCH = 16384
N_PROTO = 256
SC_ROWS = 8192
TC_ROWS = BATCH - SC_ROWS
TC_BLOCK = 1024
N_WORKERS = 32
ROWS_PER_WORKER = SC_ROWS // N_WORKERS
GROUP = 16
TILE = 32
N_TILES = ROWS_PER_WORKER // TILE
GROUPS_PER_TILE = TILE // GROUP
N_CHUNKS = N_PROTO // 16
BIG = 2**30


def _wtac_body(dist_hbm, labels_hbm, out_hbm, labels_v, tiles, out_v, sems):
    wid = lax.axis_index("c") * 16 + lax.axis_index("s")
    base = wid * ROWS_PER_WORKER

    pltpu.sync_copy(labels_hbm, labels_v)

    col_iota = lax.iota(jnp.int32, 16)

    def fire(t, b):
        pltpu.async_copy(
            dist_hbm.at[pl.ds(base + t * TILE, TILE), :], tiles[b], sems[b]
        )

    def drain(b):
        pltpu.make_async_copy(
            dist_hbm.at[pl.ds(0, TILE), :], tiles[b], sems[b]
        ).wait()

    for b in range(4):
        fire(b, b)

    def outer(o, carry):
        for b in range(4):
            t = 4 * o + b
            drain(b)
            tile = tiles[b]

            def group_body(gg, c2):
                r0 = gg * GROUP

                def half_body(h, labvec):
                    rbase = r0 + h * 8
                    for r in range(8):
                        row = rbase + r
                        best0 = tile[row, pl.ds(0, 16)]
                        bidx0 = jnp.zeros((16,), jnp.int32)
                        best1 = tile[row, pl.ds(128, 16)]
                        bidx1 = jnp.full((16,), 8, jnp.int32)
                        for j in range(1, 8):
                            v0 = tile[row, pl.ds(j * 16, 16)]
                            p0 = v0 < best0
                            best0 = jnp.where(p0, v0, best0)
                            bidx0 = jnp.where(
                                p0, jnp.full((16,), j, jnp.int32), bidx0
                            )
                            v1 = tile[row, pl.ds(128 + j * 16, 16)]
                            p1 = v1 < best1
                            best1 = jnp.where(p1, v1, best1)
                            bidx1 = jnp.where(
                                p1, jnp.full((16,), j + 8, jnp.int32), bidx1
                            )
                        pm = best1 < best0
                        best = jnp.where(pm, best1, best0)
                        bidx = jnp.where(pm, bidx1, bidx0)
                        m = jnp.min(best)
                        cand = jnp.where(
                            best == m,
                            bidx * 16 + col_iota,
                            jnp.full((16,), BIG, jnp.int32),
                        )
                        w = jnp.min(cand)
                        labvec = jnp.where(col_iota == (h * 8 + r), w, labvec)
                    return labvec

                labvec = lax.fori_loop(
                    0, 2, half_body, jnp.zeros((16,), jnp.int32)
                )
                out_v[pl.ds(t * TILE + r0, GROUP)] = plsc.load_gather(
                    labels_v, [labvec]
                )
                return c2

            lax.fori_loop(0, GROUPS_PER_TILE, group_body, 0)

            @pl.when(t + 4 < N_TILES)
            def _():
                fire(t + 4, b)
        return carry

    lax.fori_loop(0, N_TILES // 4, outer, 0)
    pltpu.sync_copy(out_v, out_hbm.at[pl.ds(base, ROWS_PER_WORKER)])


def _wtac(distances, labels):
    mesh = plsc.VectorSubcoreMesh(core_axis_name="c", subcore_axis_name="s")
    run = pl.kernel(
        _wtac_body,
        out_type=jax.ShapeDtypeStruct((SC_ROWS,), jnp.int32),
        mesh=mesh,
        scratch_types=[
            pltpu.VMEM((N_PROTO,), jnp.int32),
            [pltpu.VMEM((TILE, N_PROTO), jnp.float32)] * 4,
            pltpu.VMEM((ROWS_PER_WORKER,), jnp.int32),
            [pltpu.SemaphoreType.DMA] * 4,
        ],
        compiler_params=pltpu.CompilerParams(
            use_tc_tiling_on_sc=True, needs_layout_passes=False,
            disable_bounds_checks=True, disable_semaphore_checks=True,
            skip_device_barrier=True
        ),
        name="wtac_sc",
    )
    return run(distances, labels)


def _tc_body(dist_ref, lab_ref, out_ref):
    d = dist_ref[...]
    m = jnp.min(d, axis=1, keepdims=True)
    col = lax.broadcasted_iota(jnp.int32, d.shape, 1)
    cand = jnp.where(d == m, col, jnp.full(d.shape, BIG, jnp.int32))
    w = jnp.min(cand, axis=1, keepdims=True)
    sel = jnp.where(col == w, lab_ref[...], 0)
    out_ref[...] = jnp.sum(sel, axis=1)


def _wtac_tc(distances, labels2d):
    return pl.pallas_call(
        _tc_body,
        grid=(TC_ROWS // TC_BLOCK,),
        in_specs=[
            pl.BlockSpec((TC_BLOCK, N_PROTO),
                         lambda i: (i + SC_ROWS // TC_BLOCK, 0)),
            pl.BlockSpec((1, N_PROTO), lambda i: (0, 0)),
        ],
        out_specs=pl.BlockSpec((TC_BLOCK,), lambda i: (i,)),
        out_shape=jax.ShapeDtypeStruct((TC_ROWS,), jnp.int32),
        name="wtac_tc",
    )(distances, labels2d)


@jax.jit
def _wtac_full(distances, labels):
    y_sc = _wtac(distances, labels)
    y_tc = _wtac_tc(distances, labels.reshape(1, N_PROTO))
    return jnp.concatenate([y_sc, y_tc])


def kernel(distances, prototype_labels):
    labels = prototype_labels.astype(jnp.int32)
    return _wtac_full(distances, labels)
